# Initial kernel scaffold; baseline (speedup 1.0000x reference)
import jax
import jax.numpy as jnp
from jax.experimental import pallas as pl


def kernel(cls_preds, bbox_preds, anchors, boxes, labels):
    raise NotImplementedError("write your pallas kernel here")


# speedup vs baseline: 1.1587x; 1.1587x over previous
import jax
import jax.numpy as jnp
from jax.experimental import pallas as pl
from jax.experimental.pallas import tpu as pltpu

_B, _N, _M, _C = 4, 120000, 32, 80
_TN = 8000
_NB = _N // _TN

_INTERPRET = False


def _loss_kernel(cls_ref, bbox_ref, anc_ref, boxt_ref, lab_ref, out_ref, acc_ref):
    b = pl.program_id(0)
    i = pl.program_id(1)

    @pl.when(i == 0)
    def _init():
        acc_ref[b, 0] = 0.0
        acc_ref[b, 1] = 0.0
        acc_ref[b, 2] = 0.0

    anc = anc_ref[0]
    boxt = boxt_ref[0]
    lab = lab_ref[0]

    ax0 = anc[:, 0:1]
    ay0 = anc[:, 1:2]
    ax1 = anc[:, 2:3]
    ay1 = anc[:, 3:4]
    bx0 = boxt[0:1, :]
    by0 = boxt[1:2, :]
    bx1 = boxt[2:3, :]
    by1 = boxt[3:4, :]

    iw = jnp.maximum(jnp.minimum(ax1, bx1) - jnp.maximum(ax0, bx0), 0.0)
    ih = jnp.maximum(jnp.minimum(ay1, by1) - jnp.maximum(ay0, by0), 0.0)
    inter = iw * ih
    area_a = (ax1 - ax0) * (ay1 - ay0)
    area_b = (bx1 - bx0) * (by1 - by0)
    union = area_a + area_b - inter
    iou = inter / jnp.maximum(union, 1e-9)

    vals = jnp.max(iou, axis=1, keepdims=True)
    ids = jax.lax.broadcasted_iota(jnp.int32, (_TN, _M), 1)
    cand = jnp.where(iou >= vals, ids, _M)
    idx = jnp.min(cand, axis=1, keepdims=True)
    onehot = (ids == idx).astype(jnp.float32)

    pos = vals >= 0.5
    posf = pos.astype(jnp.float32)
    ignore = jnp.logical_and(vals >= 0.4, vals < 0.5)
    clas_maskf = jnp.where(ignore, 0.0, 1.0)

    sx0 = jnp.sum(onehot * bx0, axis=1, keepdims=True)
    sy0 = jnp.sum(onehot * by0, axis=1, keepdims=True)
    sx1 = jnp.sum(onehot * bx1, axis=1, keepdims=True)
    sy1 = jnp.sum(onehot * by1, axis=1, keepdims=True)
    slab = jnp.sum(onehot * lab, axis=1, keepdims=True)

    scx = (sx0 + sx1) * 0.5
    scy = (sy0 + sy1) * 0.5
    sw = sx1 - sx0
    sh = sy1 - sy0
    acx = (ax0 + ax1) * 0.5
    acy = (ay0 + ay1) * 0.5
    aw = jnp.maximum(ax1 - ax0, 1e-9)
    ah = jnp.maximum(ay1 - ay0, 1e-9)
    tx = ((scx - acx) / aw) / 0.1
    ty = ((scy - acy) / ah) / 0.1
    tw = jnp.log(jnp.maximum(sw, 1e-9) / aw) / 0.2
    th = jnp.log(jnp.maximum(sh, 1e-9) / ah) / 0.2

    bp = bbox_ref[0]
    d0 = bp[:, 0:1] - tx
    d1 = bp[:, 1:2] - ty
    d2 = bp[:, 2:3] - tw
    d3 = bp[:, 3:4] - th
    sl1 = 0.0
    for d in (d0, d1, d2, d3):
        ad = jnp.abs(d)
        sl1 = sl1 + jnp.where(ad < 1.0, 0.5 * d * d, ad - 0.5)
    bb_par = jnp.sum(sl1 * posf)

    x = cls_ref[0]
    cio = jax.lax.broadcasted_iota(jnp.int32, (_TN, _C), 1).astype(jnp.float32)
    t = jnp.where(cio + 1.0 == slab, posf, 0.0)
    e = jnp.exp(-jnp.abs(x))
    r = 1.0 / (1.0 + e)
    ps = jnp.where(x >= 0.0, r, 1.0 - r)
    wgt = ps + t * (1.0 - 2.0 * ps)
    alph = 0.25 + 0.5 * t
    bce = jnp.maximum(x, 0.0) - x * t + jnp.log1p(e)
    fl = wgt * wgt * alph * bce
    foc_par = jnp.sum(fl * clas_maskf)
    np_par = jnp.sum(posf)

    acc_ref[b, 0] = acc_ref[b, 0] + foc_par
    acc_ref[b, 1] = acc_ref[b, 1] + bb_par
    acc_ref[b, 2] = acc_ref[b, 2] + np_par

    @pl.when(jnp.logical_and(b == _B - 1, i == _NB - 1))
    def _fin():
        cl = 0.0
        rl = 0.0
        for bb in range(_B):
            npos = acc_ref[bb, 2]
            cl = cl + acc_ref[bb, 0] / jnp.maximum(npos, 1.0)
            rl = rl + acc_ref[bb, 1] / jnp.maximum(npos * 4.0, 1.0)
        out_ref[0, 0] = cl / _B
        out_ref[0, 1] = rl / _B


def kernel(cls_preds, bbox_preds, anchors, boxes, labels):
    boxes_t = jnp.transpose(boxes, (0, 2, 1))
    labels_f = labels.astype(jnp.float32).reshape(_B, 1, _M)

    out = pl.pallas_call(
        _loss_kernel,
        grid=(_B, _NB),
        in_specs=[
            pl.BlockSpec((1, _TN, _C), lambda b, i: (b, i, 0)),
            pl.BlockSpec((1, _TN, 4), lambda b, i: (b, i, 0)),
            pl.BlockSpec((1, _TN, 4), lambda b, i: (b, i, 0)),
            pl.BlockSpec((1, 4, _M), lambda b, i: (b, 0, 0)),
            pl.BlockSpec((1, 1, _M), lambda b, i: (b, 0, 0)),
        ],
        out_specs=pl.BlockSpec((1, 2), lambda b, i: (0, 0), memory_space=pltpu.SMEM),
        out_shape=jax.ShapeDtypeStruct((1, 2), jnp.float32),
        scratch_shapes=[pltpu.SMEM((_B, 3), jnp.float32)],
        interpret=_INTERPRET,
    )(cls_preds, bbox_preds, anchors, boxes_t, labels_f)
    return out[0, 0], out[0, 1]


# speedup vs baseline: 4.3325x; 3.7391x over previous
import jax
import jax.numpy as jnp
from jax.experimental import pallas as pl
from jax.experimental.pallas import tpu as pltpu

_B, _N, _M, _C = 4, 120000, 32, 80
_TN = 8000
_NB = _N // _TN

_INTERPRET = False


def _loss_kernel(cls_ref, bbox_ref, anc_ref, box_ref, lab_ref, out_ref, acc_ref):
    b = pl.program_id(0)
    i = pl.program_id(1)

    @pl.when(i == 0)
    def _init():
        acc_ref[b, 0] = 0.0
        acc_ref[b, 1] = 0.0
        acc_ref[b, 2] = 0.0

    anc = anc_ref[0]
    ax0 = anc[0:1, :]
    ay0 = anc[1:2, :]
    ax1 = anc[2:3, :]
    ay1 = anc[3:4, :]

    boxes_blk = box_ref[0]
    bx0 = boxes_blk[:, 0:1]
    by0 = boxes_blk[:, 1:2]
    bx1 = boxes_blk[:, 2:3]
    by1 = boxes_blk[:, 3:4]
    lab = lab_ref[0]

    iw = jnp.maximum(jnp.minimum(ax1, bx1) - jnp.maximum(ax0, bx0), 0.0)
    ih = jnp.maximum(jnp.minimum(ay1, by1) - jnp.maximum(ay0, by0), 0.0)
    inter = iw * ih
    area_a = (ax1 - ax0) * (ay1 - ay0)
    area_b = (bx1 - bx0) * (by1 - by0)
    union = area_a + area_b - inter
    iou = inter / jnp.maximum(union, 1e-9)

    vals = jnp.max(iou, axis=0, keepdims=True)
    ids = jax.lax.broadcasted_iota(jnp.int32, (_M, _TN), 0)
    cand = jnp.where(iou >= vals, ids, _M)
    idxm = jnp.min(cand, axis=0, keepdims=True)
    onehot = (ids == idxm).astype(jnp.float32)

    pos = vals >= 0.5
    posf = pos.astype(jnp.float32)
    ignore = jnp.logical_and(vals >= 0.4, vals < 0.5)
    maskf = jnp.where(ignore, 0.0, 1.0)

    sx0 = jnp.sum(onehot * bx0, axis=0, keepdims=True)
    sy0 = jnp.sum(onehot * by0, axis=0, keepdims=True)
    sx1 = jnp.sum(onehot * bx1, axis=0, keepdims=True)
    sy1 = jnp.sum(onehot * by1, axis=0, keepdims=True)
    slab = jnp.sum(onehot * lab, axis=0, keepdims=True)

    scx = (sx0 + sx1) * 0.5
    scy = (sy0 + sy1) * 0.5
    sw = sx1 - sx0
    sh = sy1 - sy0
    acx = (ax0 + ax1) * 0.5
    acy = (ay0 + ay1) * 0.5
    aw = jnp.maximum(ax1 - ax0, 1e-9)
    ah = jnp.maximum(ay1 - ay0, 1e-9)
    tx = ((scx - acx) / aw) / 0.1
    ty = ((scy - acy) / ah) / 0.1
    tw = jnp.log(jnp.maximum(sw, 1e-9) / aw) / 0.2
    th = jnp.log(jnp.maximum(sh, 1e-9) / ah) / 0.2

    bp = bbox_ref[0]
    sl1 = jnp.zeros((1, _TN), jnp.float32)
    for k, enc in enumerate((tx, ty, tw, th)):
        d = bp[k:k + 1, :] - enc
        ad = jnp.abs(d)
        sl1 = sl1 + jnp.where(ad < 1.0, 0.5 * d * d, ad - 0.5)
    bb_par = jnp.sum(sl1 * posf)
    np_par = jnp.sum(posf)

    xt = jax.lax.transpose(cls_ref[0], (1, 0))
    ciof = jax.lax.broadcasted_iota(jnp.int32, (_C, _TN), 0).astype(jnp.float32)
    tgt = jnp.where(jnp.logical_and(ciof == slab - 1.0, pos), 1.0, 0.0)
    e = jnp.exp(-jnp.abs(xt))
    r = 1.0 / (1.0 + e)
    ps = jnp.where(xt >= 0.0, r, 1.0 - r)
    wgt = ps + tgt * (1.0 - 2.0 * ps)
    alph = 0.25 + 0.5 * tgt
    bce = jnp.maximum(xt, 0.0) - xt * tgt + jnp.log1p(e)
    fl = wgt * wgt * alph * bce
    foc_par = jnp.sum(fl * maskf)

    acc_ref[b, 0] = acc_ref[b, 0] + foc_par
    acc_ref[b, 1] = acc_ref[b, 1] + bb_par
    acc_ref[b, 2] = acc_ref[b, 2] + np_par

    @pl.when(jnp.logical_and(b == _B - 1, i == _NB - 1))
    def _fin():
        cl = 0.0
        rl = 0.0
        for bb in range(_B):
            npos = acc_ref[bb, 2]
            cl = cl + acc_ref[bb, 0] / jnp.maximum(npos, 1.0)
            rl = rl + acc_ref[bb, 1] / jnp.maximum(npos * 4.0, 1.0)
        out_ref[0, 0] = cl / _B
        out_ref[0, 1] = rl / _B


def _retile(a):
    a = jnp.transpose(a, (0, 2, 1))
    a = a.reshape(_B, 4, _NB, _TN)
    a = jnp.transpose(a, (0, 2, 1, 3))
    return a.reshape(_B * _NB, 4, _TN)


def kernel(cls_preds, bbox_preds, anchors, boxes, labels):
    anc_r = _retile(anchors)
    bbox_r = _retile(bbox_preds)
    labels_f = labels.astype(jnp.float32).reshape(_B, _M, 1)

    out = pl.pallas_call(
        _loss_kernel,
        grid=(_B, _NB),
        in_specs=[
            pl.BlockSpec((1, _TN, _C), lambda b, i: (b, i, 0)),
            pl.BlockSpec((1, 4, _TN), lambda b, i: (b * _NB + i, 0, 0)),
            pl.BlockSpec((1, 4, _TN), lambda b, i: (b * _NB + i, 0, 0)),
            pl.BlockSpec((1, _M, 4), lambda b, i: (b, 0, 0)),
            pl.BlockSpec((1, _M, 1), lambda b, i: (b, 0, 0)),
        ],
        out_specs=pl.BlockSpec((1, 2), lambda b, i: (0, 0), memory_space=pltpu.SMEM),
        out_shape=jax.ShapeDtypeStruct((1, 2), jnp.float32),
        scratch_shapes=[pltpu.SMEM((_B, 3), jnp.float32)],
        interpret=_INTERPRET,
    )(cls_preds, bbox_r, anc_r, boxes, labels_f)
    return out[0, 0], out[0, 1]


# speedup vs baseline: 5.0503x; 1.1657x over previous
import jax
import jax.numpy as jnp
from jax.experimental import pallas as pl
from jax.experimental.pallas import tpu as pltpu

_B, _N, _M, _C = 4, 120000, 32, 80
_TN = 8000
_NB = _N // _TN

_INTERPRET = False


def _loss_kernel(cls_ref, bbox_ref, anc_ref, box_ref, ext_ref, out_ref, acc_ref):
    b = pl.program_id(0)
    i = pl.program_id(1)

    @pl.when(i == 0)
    def _init():
        acc_ref[b, 0] = 0.0
        acc_ref[b, 1] = 0.0
        acc_ref[b, 2] = 0.0

    anc = anc_ref[0]
    ax0 = anc[0:1, :]
    ay0 = anc[1:2, :]
    ax1 = anc[2:3, :]
    ay1 = anc[3:4, :]

    boxes_blk = box_ref[0]
    bx0 = boxes_blk[:, 0:1]
    by0 = boxes_blk[:, 1:2]
    bx1 = boxes_blk[:, 2:3]
    by1 = boxes_blk[:, 3:4]
    ext = ext_ref[0]

    iw = jnp.maximum(jnp.minimum(ax1, bx1) - jnp.maximum(ax0, bx0), 0.0)
    ih = jnp.maximum(jnp.minimum(ay1, by1) - jnp.maximum(ay0, by0), 0.0)
    inter = iw * ih
    area_a = (ax1 - ax0) * (ay1 - ay0)
    area_b = (bx1 - bx0) * (by1 - by0)
    union = area_a + area_b - inter
    iou = inter / jnp.maximum(union, 1e-9)

    vals = jnp.max(iou, axis=0, keepdims=True)
    ids = jax.lax.broadcasted_iota(jnp.int32, (_M, _TN), 0)
    cand = jnp.where(iou >= vals, ids, _M)
    idxm = jnp.min(cand, axis=0, keepdims=True)
    onehot = (ids == idxm).astype(jnp.float32)

    pos = vals >= 0.5
    posf = pos.astype(jnp.float32)
    ignore = jnp.logical_and(vals >= 0.4, vals < 0.5)
    maskf = jnp.where(ignore, 0.0, 1.0)

    sel8 = jax.lax.dot_general(ext, onehot, (((1,), (0,)), ((), ())),
                               preferred_element_type=jnp.float32)
    sx0 = sel8[0:1, :]
    sy0 = sel8[1:2, :]
    sx1 = sel8[2:3, :]
    sy1 = sel8[3:4, :]
    slab = sel8[4:5, :]

    scx = (sx0 + sx1) * 0.5
    scy = (sy0 + sy1) * 0.5
    sw = sx1 - sx0
    sh = sy1 - sy0
    acx = (ax0 + ax1) * 0.5
    acy = (ay0 + ay1) * 0.5
    aw = jnp.maximum(ax1 - ax0, 1e-9)
    ah = jnp.maximum(ay1 - ay0, 1e-9)
    tx = ((scx - acx) / aw) / 0.1
    ty = ((scy - acy) / ah) / 0.1
    tw = jnp.log(jnp.maximum(sw, 1e-9) / aw) / 0.2
    th = jnp.log(jnp.maximum(sh, 1e-9) / ah) / 0.2

    bp = bbox_ref[0]
    sl1 = jnp.zeros((1, _TN), jnp.float32)
    for k, enc in enumerate((tx, ty, tw, th)):
        d = bp[k:k + 1, :] - enc
        ad = jnp.abs(d)
        sl1 = sl1 + jnp.where(ad < 1.0, 0.5 * d * d, ad - 0.5)
    bb_par = jnp.sum(sl1 * posf)
    np_par = jnp.sum(posf)

    xt = jax.lax.transpose(cls_ref[0], (1, 0))
    ciof = jax.lax.broadcasted_iota(jnp.int32, (_C, _TN), 0).astype(jnp.float32)
    e = jnp.exp(-jnp.abs(xt))
    r = 1.0 / (1.0 + e)
    ps = jnp.where(xt >= 0.0, r, 1.0 - r)
    sp = jnp.maximum(xt, 0.0) + jnp.log1p(e)
    f0 = ps * ps * sp
    f0sum = jnp.sum(f0 * maskf)
    xl = jnp.sum(jnp.where(ciof == slab - 1.0, xt, 0.0), axis=0,
                 keepdims=True)
    el = jnp.exp(-jnp.abs(xl))
    rl = 1.0 / (1.0 + el)
    psl = jnp.where(xl >= 0.0, rl, 1.0 - rl)
    spl = jnp.maximum(xl, 0.0) + jnp.log1p(el)
    f0l = 0.25 * psl * psl * spl
    f1l = 0.75 * (1.0 - psl) * (1.0 - psl) * (spl - xl)
    foc_par = 0.25 * f0sum + jnp.sum((f1l - f0l) * posf)

    acc_ref[b, 0] = acc_ref[b, 0] + foc_par
    acc_ref[b, 1] = acc_ref[b, 1] + bb_par
    acc_ref[b, 2] = acc_ref[b, 2] + np_par

    @pl.when(jnp.logical_and(b == _B - 1, i == _NB - 1))
    def _fin():
        cl = 0.0
        rl = 0.0
        for bb in range(_B):
            npos = acc_ref[bb, 2]
            cl = cl + acc_ref[bb, 0] / jnp.maximum(npos, 1.0)
            rl = rl + acc_ref[bb, 1] / jnp.maximum(npos * 4.0, 1.0)
        out_ref[0, 0] = cl / _B
        out_ref[0, 1] = rl / _B


def _retile(a):
    a = jnp.transpose(a, (0, 2, 1))
    a = a.reshape(_B, 4, _NB, _TN)
    a = jnp.transpose(a, (0, 2, 1, 3))
    return a.reshape(_B * _NB, 4, _TN)


def kernel(cls_preds, bbox_preds, anchors, boxes, labels):
    anc_r = _retile(anchors)
    bbox_r = _retile(bbox_preds)
    ext = jnp.concatenate(
        [jnp.transpose(boxes, (0, 2, 1)),
         labels.astype(jnp.float32)[:, None, :],
         jnp.zeros((_B, 3, _M), jnp.float32)], axis=1)

    out = pl.pallas_call(
        _loss_kernel,
        grid=(_B, _NB),
        in_specs=[
            pl.BlockSpec((1, _TN, _C), lambda b, i: (b, i, 0)),
            pl.BlockSpec((1, 4, _TN), lambda b, i: (b * _NB + i, 0, 0)),
            pl.BlockSpec((1, 4, _TN), lambda b, i: (b * _NB + i, 0, 0)),
            pl.BlockSpec((1, _M, 4), lambda b, i: (b, 0, 0)),
            pl.BlockSpec((1, 8, _M), lambda b, i: (b, 0, 0)),
        ],
        out_specs=pl.BlockSpec((1, 2), lambda b, i: (0, 0), memory_space=pltpu.SMEM),
        out_shape=jax.ShapeDtypeStruct((1, 2), jnp.float32),
        scratch_shapes=[pltpu.SMEM((_B, 3), jnp.float32)],
        interpret=_INTERPRET,
    )(cls_preds, bbox_r, anc_r, boxes, ext)
    return out[0, 0], out[0, 1]


# speedup vs baseline: 5.1108x; 1.0120x over previous
import jax
import jax.numpy as jnp
from jax.experimental import pallas as pl
from jax.experimental.pallas import tpu as pltpu

_B, _N, _M, _C = 4, 120000, 32, 80
_TN = 15000
_NB = _N // _TN

_INTERPRET = False


def _loss_kernel(cls_ref, bbox_ref, anc_ref, box_ref, ext_ref, out_ref, acc_ref):
    b = pl.program_id(0)
    i = pl.program_id(1)

    @pl.when(i == 0)
    def _init():
        acc_ref[b, 0] = 0.0
        acc_ref[b, 1] = 0.0
        acc_ref[b, 2] = 0.0

    anc = anc_ref[0]
    ax0 = anc[0:1, :]
    ay0 = anc[1:2, :]
    ax1 = anc[2:3, :]
    ay1 = anc[3:4, :]

    boxes_blk = box_ref[0]
    bx0 = boxes_blk[:, 0:1]
    by0 = boxes_blk[:, 1:2]
    bx1 = boxes_blk[:, 2:3]
    by1 = boxes_blk[:, 3:4]
    ext = ext_ref[0]

    iw = jnp.maximum(jnp.minimum(ax1, bx1) - jnp.maximum(ax0, bx0), 0.0)
    ih = jnp.maximum(jnp.minimum(ay1, by1) - jnp.maximum(ay0, by0), 0.0)
    inter = iw * ih
    area_a = (ax1 - ax0) * (ay1 - ay0)
    area_b = (bx1 - bx0) * (by1 - by0)
    union = area_a + area_b - inter
    iou = inter / jnp.maximum(union, 1e-9)

    vals = jnp.max(iou, axis=0, keepdims=True)
    ids = jax.lax.broadcasted_iota(jnp.int32, (_M, _TN), 0)
    cand = jnp.where(iou >= vals, ids, _M)
    idxm = jnp.min(cand, axis=0, keepdims=True)
    onehot = (ids == idxm).astype(jnp.float32)

    pos = vals >= 0.5
    posf = pos.astype(jnp.float32)
    ignore = jnp.logical_and(vals >= 0.4, vals < 0.5)
    maskf = jnp.where(ignore, 0.0, 1.0)

    sel8 = jax.lax.dot_general(ext, onehot, (((1,), (0,)), ((), ())),
                               preferred_element_type=jnp.float32)
    sx0 = sel8[0:1, :]
    sy0 = sel8[1:2, :]
    sx1 = sel8[2:3, :]
    sy1 = sel8[3:4, :]
    slab = sel8[4:5, :]

    scx = (sx0 + sx1) * 0.5
    scy = (sy0 + sy1) * 0.5
    sw = sx1 - sx0
    sh = sy1 - sy0
    acx = (ax0 + ax1) * 0.5
    acy = (ay0 + ay1) * 0.5
    aw = jnp.maximum(ax1 - ax0, 1e-9)
    ah = jnp.maximum(ay1 - ay0, 1e-9)
    tx = ((scx - acx) / aw) / 0.1
    ty = ((scy - acy) / ah) / 0.1
    tw = jnp.log(jnp.maximum(sw, 1e-9) / aw) / 0.2
    th = jnp.log(jnp.maximum(sh, 1e-9) / ah) / 0.2

    bp = bbox_ref[0]
    sl1 = jnp.zeros((1, _TN), jnp.float32)
    for k, enc in enumerate((tx, ty, tw, th)):
        d = bp[k:k + 1, :] - enc
        ad = jnp.abs(d)
        sl1 = sl1 + jnp.where(ad < 1.0, 0.5 * d * d, ad - 0.5)
    bb_par = jnp.sum(sl1 * posf)
    np_par = jnp.sum(posf)

    xt = jax.lax.transpose(cls_ref[0], (1, 0))
    ciof = jax.lax.broadcasted_iota(jnp.int32, (_C, _TN), 0).astype(jnp.float32)
    e = jnp.exp(-jnp.abs(xt))
    r = 1.0 / (1.0 + e)
    ps = jnp.where(xt >= 0.0, r, 1.0 - r)
    sp = jnp.maximum(xt, 0.0) + jnp.log1p(e)
    f0 = ps * ps * sp
    f0sum = jnp.sum(f0 * maskf)
    xl = jnp.sum(jnp.where(ciof == slab - 1.0, xt, 0.0), axis=0,
                 keepdims=True)
    el = jnp.exp(-jnp.abs(xl))
    rl = 1.0 / (1.0 + el)
    psl = jnp.where(xl >= 0.0, rl, 1.0 - rl)
    spl = jnp.maximum(xl, 0.0) + jnp.log1p(el)
    f0l = 0.25 * psl * psl * spl
    f1l = 0.75 * (1.0 - psl) * (1.0 - psl) * (spl - xl)
    foc_par = 0.25 * f0sum + jnp.sum((f1l - f0l) * posf)

    acc_ref[b, 0] = acc_ref[b, 0] + foc_par
    acc_ref[b, 1] = acc_ref[b, 1] + bb_par
    acc_ref[b, 2] = acc_ref[b, 2] + np_par

    @pl.when(jnp.logical_and(b == _B - 1, i == _NB - 1))
    def _fin():
        cl = 0.0
        rl = 0.0
        for bb in range(_B):
            npos = acc_ref[bb, 2]
            cl = cl + acc_ref[bb, 0] / jnp.maximum(npos, 1.0)
            rl = rl + acc_ref[bb, 1] / jnp.maximum(npos * 4.0, 1.0)
        out_ref[0, 0] = cl / _B
        out_ref[0, 1] = rl / _B


def _retile(a):
    a = jnp.transpose(a, (0, 2, 1))
    a = a.reshape(_B, 4, _NB, _TN)
    a = jnp.transpose(a, (0, 2, 1, 3))
    return a.reshape(_B * _NB, 4, _TN)


def kernel(cls_preds, bbox_preds, anchors, boxes, labels):
    anc_r = _retile(anchors)
    bbox_r = _retile(bbox_preds)
    ext = jnp.concatenate(
        [jnp.transpose(boxes, (0, 2, 1)),
         labels.astype(jnp.float32)[:, None, :],
         jnp.zeros((_B, 3, _M), jnp.float32)], axis=1)

    out = pl.pallas_call(
        _loss_kernel,
        grid=(_B, _NB),
        in_specs=[
            pl.BlockSpec((1, _TN, _C), lambda b, i: (b, i, 0)),
            pl.BlockSpec((1, 4, _TN), lambda b, i: (b * _NB + i, 0, 0)),
            pl.BlockSpec((1, 4, _TN), lambda b, i: (b * _NB + i, 0, 0)),
            pl.BlockSpec((1, _M, 4), lambda b, i: (b, 0, 0)),
            pl.BlockSpec((1, 8, _M), lambda b, i: (b, 0, 0)),
        ],
        out_specs=pl.BlockSpec((1, 2), lambda b, i: (0, 0), memory_space=pltpu.SMEM),
        out_shape=jax.ShapeDtypeStruct((1, 2), jnp.float32),
        scratch_shapes=[pltpu.SMEM((_B, 3), jnp.float32)],
        interpret=_INTERPRET,
    )(cls_preds, bbox_r, anc_r, boxes, ext)
    return out[0, 0], out[0, 1]
